# baseline (speedup 1.0000x reference)
import functools

import jax
import jax.numpy as jnp
from jax import lax
from jax.experimental import pallas as pl
from jax.experimental.pallas import tpu as pltpu
from jax.experimental.pallas import tpu_sc as plsc

_B, _D, _N = 16, 3, 65536
_K = 1024
_L = 16
_BINS = 2048
_SHIFT = 20
_NV = _N // _L
_CAND = 2048
_PADI = 0x7FFFFFFF


def _norm_body(p1_ref, pc_ref, out_ref):
    b = pl.program_id(0)
    d = pc_ref[0]
    qx = p1_ref[b, 0]
    qy = p1_ref[b, 1]
    qz = p1_ref[b, 2]
    dx = d[0:1, :] - qx
    dy = d[1:2, :] - qy
    dz = d[2:3, :] - qz
    s = (dx * dx + dy * dy) + dz * dz
    out_ref[0] = jnp.sqrt(s)


def _norms(pcloud, P1):
    return pl.pallas_call(
        _norm_body,
        grid=(_B,),
        in_specs=[
            pl.BlockSpec((_B, _D), lambda b: (0, 0), memory_space=pltpu.SMEM),
            pl.BlockSpec((1, _D, _N), lambda b: (b, 0, 0)),
        ],
        out_specs=pl.BlockSpec((1, 1, _N), lambda b: (b, 0, 0)),
        out_shape=jax.ShapeDtypeStruct((_B, 1, _N), jnp.float32),
    )(P1, pcloud)


def _scalar(x):
    return jnp.max(x) if getattr(x, "ndim", 0) else x


def _permute(x, perm):
    dn = lax.GatherDimensionNumbers(
        offset_dims=(), collapsed_slice_dims=(0,), start_index_map=(0,))
    return lax.gather(x, perm[:, None], dimension_numbers=dn, slice_sizes=(1,),
                      mode=lax.GatherScatterMode.PROMISE_IN_BOUNDS)


def _sc_body(dist, pc, zeros_h, padf_h, padi_h, a_out, b_out,
             keys_v, hist_v, skey_v, sidx_v, bkey_v, bidx_v, aout_v, gidx_v,
             sem):
    wid = lax.axis_index("s") * 2 + lax.axis_index("c")

    @pl.when(wid < _B)
    def _():
        b = wid
        iota = lax.iota(jnp.int32, _L)
        ones = jnp.ones((_L,), jnp.int32)

        cps = [
            pltpu.async_copy(dist.at[pl.ds(b * _N, _N)], keys_v, sem),
            pltpu.async_copy(zeros_h, hist_v, sem),
            pltpu.async_copy(padf_h, skey_v, sem),
            pltpu.async_copy(padi_h, sidx_v, sem),
            pltpu.async_copy(padf_h, bkey_v, sem),
            pltpu.async_copy(padi_h, bidx_v, sem),
        ]
        for cp in cps:
            cp.wait()

        def hist_body(i, _):
            for u in range(8):
                kv = keys_v[pl.ds((i * 8 + u) * _L, _L)]
                bits = lax.bitcast_convert_type(kv, jnp.int32)
                binv = lax.shift_right_logical(bits, _SHIFT)
                addr = iota * _BINS + binv
                plsc.addupdate_scatter(hist_v, [addr], ones)
            return 0
        lax.fori_loop(0, _NV // 8, hist_body, 0)

        def red_body(j, _):
            acc = hist_v[pl.ds(j * _L, _L)]
            for r in range(1, _L):
                acc = acc + hist_v[pl.ds(r * _BINS + j * _L, _L)]
            hist_v[pl.ds(j * _L, _L)] = acc
            return 0
        lax.fori_loop(0, _BINS // _L, red_body, 0)

        def scan_body(j, carry):
            total, bstar, found = carry
            h16 = hist_v[pl.ds(j * _L, _L)]
            c16 = plsc.cumsum(h16)
            chunk = jnp.max(c16)
            cum = total + c16
            cross = cum >= _K
            crossed = jnp.logical_and(total + chunk >= _K, found == 0)
            pos = _scalar(plsc.all_reduce_ffs(cross))
            bstar = jnp.where(crossed, j * _L + pos, bstar)
            found = jnp.where(crossed, 1, found)
            return (total + chunk, bstar, found)
        _, bstar, _ = lax.fori_loop(
            0, _BINS // _L, scan_body,
            (jnp.int32(0), jnp.int32(0), jnp.int32(0)))

        def comp_body(i, carry):
            kv = keys_v[pl.ds(i * _L, _L)]
            bits = lax.bitcast_convert_type(kv, jnp.int32)
            binv = lax.shift_right_logical(bits, _SHIFT)
            mboth = binv <= bstar
            nhit = _scalar(plsc.all_reduce_population_count(mboth))

            def hit(c):
                nd, nb = c
                idx16 = i * _L + iota
                mdef = binv < bstar
                mbor = binv == bstar
                plsc.store_compressed(skey_v.at[pl.ds(nd, _L)], kv, mask=mdef)
                plsc.store_compressed(sidx_v.at[pl.ds(nd, _L)], idx16,
                                      mask=mdef)
                plsc.store_compressed(bkey_v.at[pl.ds(nb, _L)], kv, mask=mbor)
                plsc.store_compressed(bidx_v.at[pl.ds(nb, _L)], idx16,
                                      mask=mbor)
                nd = nd + _scalar(plsc.all_reduce_population_count(mdef))
                nb = nb + _scalar(plsc.all_reduce_population_count(mbor))
                nb = jnp.minimum(nb, jnp.int32(_CAND - _L))
                return (nd, nb)
            return lax.cond(nhit > 0, hit, lambda c: c, carry)
        nd, nb = lax.fori_loop(0, _NV, comp_body,
                               (jnp.int32(0), jnp.int32(0)))

        nb_c = jnp.minimum(nb, jnp.int32(_CAND) - nd)

        def app_body(i, _):
            skey_v[pl.ds(nd + i * _L, _L)] = bkey_v[pl.ds(i * _L, _L)]
            sidx_v[pl.ds(nd + i * _L, _L)] = bidx_v[pl.ds(i * _L, _L)]
            return 0
        lax.fori_loop(0, (nb_c + _L - 1) // _L, app_body, 0)

        nvec = _CAND // _L
        k = 2
        while k <= _CAND:
            j = k // 2
            while j >= 1:
                if j >= _L:
                    dd = j // _L
                    s = dd.bit_length() - 1

                    def inter_body(u, _, k=k, dd=dd, s=s):
                        v_lo = ((u >> s) << (s + 1)) | (u & (dd - 1))
                        v_hi = v_lo + dd
                        ak = skey_v[pl.ds(v_lo * _L, _L)]
                        ai = sidx_v[pl.ds(v_lo * _L, _L)]
                        bk = skey_v[pl.ds(v_hi * _L, _L)]
                        bi = sidx_v[pl.ds(v_hi * _L, _L)]
                        asc = ((v_lo * _L) & k) == 0
                        altb = jnp.logical_or(
                            ak < bk, jnp.logical_and(ak == bk, ai < bi))
                        sel = altb == jnp.broadcast_to(asc, (_L,))
                        skey_v[pl.ds(v_lo * _L, _L)] = jnp.where(sel, ak, bk)
                        sidx_v[pl.ds(v_lo * _L, _L)] = jnp.where(sel, ai, bi)
                        skey_v[pl.ds(v_hi * _L, _L)] = jnp.where(sel, bk, ak)
                        sidx_v[pl.ds(v_hi * _L, _L)] = jnp.where(sel, bi, ai)
                        return 0
                    lax.fori_loop(0, nvec // 2, inter_body, 0)
                else:
                    perm = jnp.bitwise_xor(iota, j)
                    is_hi = (iota & j) != 0

                    def intra_body(v, _, k=k, j=j, perm=perm, is_hi=is_hi):
                        ak = skey_v[pl.ds(v * _L, _L)]
                        ai = sidx_v[pl.ds(v * _L, _L)]
                        bk = _permute(ak, perm)
                        bi = _permute(ai, perm)
                        if k <= 8:
                            asc = (iota & k) == 0
                        else:
                            asc = jnp.broadcast_to(((v * _L) & k) == 0, (_L,))
                        hold_min = asc != is_hi
                        altb = jnp.logical_or(
                            ak < bk, jnp.logical_and(ak == bk, ai < bi))
                        sel = altb == hold_min
                        skey_v[pl.ds(v * _L, _L)] = jnp.where(sel, ak, bk)
                        sidx_v[pl.ds(v * _L, _L)] = jnp.where(sel, ai, bi)
                        return 0
                    lax.fori_loop(0, nvec, intra_body, 0)
                j //= 2
            k *= 2

        pltpu.sync_copy(sidx_v.at[pl.ds(0, _K)], b_out.at[pl.ds(b * _K, _K)])

        def gi_body(v, _):
            si = sidx_v[pl.ds(v * _L, _L)]
            for c in range(_D):
                gidx_v[pl.ds(c * _K + v * _L, _L)] = si + (b * _D + c) * _N
            return 0
        lax.fori_loop(0, _K // _L, gi_body, 0)
        gcps = [
            pltpu.async_copy(pc.at[gidx_v.at[pl.ds(c * _K, _K)]],
                             aout_v.at[pl.ds(c * _K, _K)], sem)
            for c in range(_D)
        ]
        for cp in gcps:
            cp.wait()
        pltpu.sync_copy(aout_v, a_out.at[pl.ds(b * _D * _K, _D * _K)])


def _sc_topk(dist, pcloud):
    mesh = plsc.VectorSubcoreMesh(core_axis_name="c", subcore_axis_name="s")
    f = pl.kernel(
        _sc_body,
        out_type=(
            jax.ShapeDtypeStruct((_B * _D * _K,), jnp.float32),
            jax.ShapeDtypeStruct((_B * _K,), jnp.int32),
        ),
        mesh=mesh,
        compiler_params=pltpu.CompilerParams(needs_layout_passes=False),
        scratch_types=[
            pltpu.VMEM((_N,), jnp.float32),
            pltpu.VMEM((_L * _BINS,), jnp.int32),
            pltpu.VMEM((_CAND + _L,), jnp.float32),
            pltpu.VMEM((_CAND + _L,), jnp.int32),
            pltpu.VMEM((_CAND + _L,), jnp.float32),
            pltpu.VMEM((_CAND + _L,), jnp.int32),
            pltpu.VMEM((_D * _K,), jnp.float32),
            pltpu.VMEM((_D * _K,), jnp.int32),
            pltpu.SemaphoreType.DMA,
        ],
    )
    zeros_h = jnp.zeros((_L * _BINS,), jnp.int32)
    padf_h = jnp.full((_CAND + _L,), jnp.inf, jnp.float32)
    padi_h = jnp.full((_CAND + _L,), _PADI, jnp.int32)
    a_f, b_f = f(dist.reshape(_B * _N), pcloud.reshape(_B * _D * _N),
                 zeros_h, padf_h, padi_h)
    return a_f.reshape(_B, _D, _K), b_f.reshape(_B, _K)


def kernel(pcloud, P1, K):
    dist = _norms(pcloud, P1).reshape(_B, _N)
    a, idx = _sc_topk(dist, pcloud)
    off = (jnp.asarray(K) - _K).astype(jnp.int32)
    return (a, idx + off)


# speedup vs baseline: 1.2120x; 1.2120x over previous
import functools

import jax
import jax.numpy as jnp
from jax import lax
from jax.experimental import pallas as pl
from jax.experimental.pallas import tpu as pltpu
from jax.experimental.pallas import tpu_sc as plsc

_B, _D, _N = 16, 3, 65536
_K = 1024
_L = 16
_BINS = 2048
_SHIFT = 20
_NV = _N // _L
_CAND = 2048
_PADI = 0x7FFFFFFF


def _norm_body(p1_ref, pc_ref, out_ref):
    b = pl.program_id(0)
    d = pc_ref[0]
    qx = p1_ref[b, 0]
    qy = p1_ref[b, 1]
    qz = p1_ref[b, 2]
    dx = d[0:1, :] - qx
    dy = d[1:2, :] - qy
    dz = d[2:3, :] - qz
    s = (dx * dx + dz * dz) + dy * dy
    out_ref[0] = jnp.sqrt(s)


def _norms(pcloud, P1):
    return pl.pallas_call(
        _norm_body,
        grid=(_B,),
        in_specs=[
            pl.BlockSpec((_B, _D), lambda b: (0, 0), memory_space=pltpu.SMEM),
            pl.BlockSpec((1, _D, _N), lambda b: (b, 0, 0)),
        ],
        out_specs=pl.BlockSpec((1, 1, _N), lambda b: (b, 0, 0)),
        out_shape=jax.ShapeDtypeStruct((_B, 1, _N), jnp.float32),
    )(P1, pcloud)


def _scalar(x):
    return jnp.max(x) if getattr(x, "ndim", 0) else x


def _permute(x, perm):
    dn = lax.GatherDimensionNumbers(
        offset_dims=(), collapsed_slice_dims=(0,), start_index_map=(0,))
    return lax.gather(x, perm[:, None], dimension_numbers=dn, slice_sizes=(1,),
                      mode=lax.GatherScatterMode.PROMISE_IN_BOUNDS)


def _sc_body(dist, pc, zeros_h, padf_h, padi_h, a_out, b_out,
             keys_v, hist_v, skey_v, sidx_v, bkey_v, bidx_v, aout_v, gidx_v,
             sem):
    wid = lax.axis_index("s") * 2 + lax.axis_index("c")

    @pl.when(wid < _B)
    def _():
        b = wid
        iota = lax.iota(jnp.int32, _L)
        ones = jnp.ones((_L,), jnp.int32)

        cps = [
            pltpu.async_copy(dist.at[pl.ds(b * _N, _N)], keys_v, sem),
            pltpu.async_copy(zeros_h, hist_v, sem),
            pltpu.async_copy(padf_h, skey_v, sem),
            pltpu.async_copy(padi_h, sidx_v, sem),
            pltpu.async_copy(padf_h, bkey_v, sem),
            pltpu.async_copy(padi_h, bidx_v, sem),
        ]
        for cp in cps:
            cp.wait()

        def hist_body(i, _):
            for u in range(8):
                kv = keys_v[pl.ds((i * 8 + u) * _L, _L)]
                bits = lax.bitcast_convert_type(kv, jnp.int32)
                binv = lax.shift_right_logical(bits, _SHIFT)
                addr = iota * _BINS + binv
                plsc.addupdate_scatter(hist_v, [addr], ones)
            return 0
        lax.fori_loop(0, _NV // 8, hist_body, 0)

        def red_body(j, _):
            acc = hist_v[pl.ds(j * _L, _L)]
            for r in range(1, _L):
                acc = acc + hist_v[pl.ds(r * _BINS + j * _L, _L)]
            hist_v[pl.ds(j * _L, _L)] = acc
            return 0
        lax.fori_loop(0, _BINS // _L, red_body, 0)

        def scan_body(j, carry):
            total, bstar, found = carry
            h16 = hist_v[pl.ds(j * _L, _L)]
            c16 = plsc.cumsum(h16)
            chunk = jnp.max(c16)
            cum = total + c16
            cross = cum >= _K
            crossed = jnp.logical_and(total + chunk >= _K, found == 0)
            pos = _scalar(plsc.all_reduce_ffs(cross))
            bstar = jnp.where(crossed, j * _L + pos, bstar)
            found = jnp.where(crossed, 1, found)
            return (total + chunk, bstar, found)
        _, bstar, _ = lax.fori_loop(
            0, _BINS // _L, scan_body,
            (jnp.int32(0), jnp.int32(0), jnp.int32(0)))

        def comp_body(i, carry):
            nd, nb = carry
            kv = keys_v[pl.ds(i * _L, _L)]
            bits = lax.bitcast_convert_type(kv, jnp.int32)
            binv = lax.shift_right_logical(bits, _SHIFT)
            idx16 = i * _L + iota
            mdef = binv < bstar
            mbor = binv == bstar
            plsc.store_compressed(skey_v.at[pl.ds(nd, _L)], kv, mask=mdef)
            plsc.store_compressed(sidx_v.at[pl.ds(nd, _L)], idx16, mask=mdef)
            plsc.store_compressed(bkey_v.at[pl.ds(nb, _L)], kv, mask=mbor)
            plsc.store_compressed(bidx_v.at[pl.ds(nb, _L)], idx16, mask=mbor)
            nd = nd + _scalar(plsc.all_reduce_population_count(mdef))
            nb = nb + _scalar(plsc.all_reduce_population_count(mbor))
            nb = jnp.minimum(nb, jnp.int32(_CAND - _L))
            return (nd, nb)
        nd, nb = lax.fori_loop(0, _NV, comp_body,
                               (jnp.int32(0), jnp.int32(0)))

        nb_c = jnp.minimum(nb, jnp.int32(_CAND) - nd)

        def app_body(i, _):
            skey_v[pl.ds(nd + i * _L, _L)] = bkey_v[pl.ds(i * _L, _L)]
            sidx_v[pl.ds(nd + i * _L, _L)] = bidx_v[pl.ds(i * _L, _L)]
            return 0
        lax.fori_loop(0, (nb_c + _L - 1) // _L, app_body, 0)

        nvec = _CAND // _L
        k = 2
        while k <= _CAND:
            j = k // 2
            while j >= 1:
                if j >= _L:
                    dd = j // _L
                    s = dd.bit_length() - 1

                    def inter_body(u, _, k=k, dd=dd, s=s):
                        v_lo = ((u >> s) << (s + 1)) | (u & (dd - 1))
                        v_hi = v_lo + dd
                        ak = skey_v[pl.ds(v_lo * _L, _L)]
                        ai = sidx_v[pl.ds(v_lo * _L, _L)]
                        bk = skey_v[pl.ds(v_hi * _L, _L)]
                        bi = sidx_v[pl.ds(v_hi * _L, _L)]
                        asc = ((v_lo * _L) & k) == 0
                        altb = jnp.logical_or(
                            ak < bk, jnp.logical_and(ak == bk, ai < bi))
                        sel = altb == jnp.broadcast_to(asc, (_L,))
                        skey_v[pl.ds(v_lo * _L, _L)] = jnp.where(sel, ak, bk)
                        sidx_v[pl.ds(v_lo * _L, _L)] = jnp.where(sel, ai, bi)
                        skey_v[pl.ds(v_hi * _L, _L)] = jnp.where(sel, bk, ak)
                        sidx_v[pl.ds(v_hi * _L, _L)] = jnp.where(sel, bi, ai)
                        return 0
                    lax.fori_loop(0, nvec // 2, inter_body, 0)
                else:
                    perm = jnp.bitwise_xor(iota, j)
                    is_hi = (iota & j) != 0

                    def intra_body(v, _, k=k, j=j, perm=perm, is_hi=is_hi):
                        ak = skey_v[pl.ds(v * _L, _L)]
                        ai = sidx_v[pl.ds(v * _L, _L)]
                        bk = _permute(ak, perm)
                        bi = _permute(ai, perm)
                        if k <= 8:
                            asc = (iota & k) == 0
                        else:
                            asc = jnp.broadcast_to(((v * _L) & k) == 0, (_L,))
                        hold_min = asc != is_hi
                        altb = jnp.logical_or(
                            ak < bk, jnp.logical_and(ak == bk, ai < bi))
                        sel = altb == hold_min
                        skey_v[pl.ds(v * _L, _L)] = jnp.where(sel, ak, bk)
                        sidx_v[pl.ds(v * _L, _L)] = jnp.where(sel, ai, bi)
                        return 0
                    lax.fori_loop(0, nvec, intra_body, 0)
                j //= 2
            k *= 2

        pltpu.sync_copy(sidx_v.at[pl.ds(0, _K)], b_out.at[pl.ds(b * _K, _K)])

        def gi_body(v, _):
            si = sidx_v[pl.ds(v * _L, _L)]
            for c in range(_D):
                gidx_v[pl.ds(c * _K + v * _L, _L)] = si + (b * _D + c) * _N
            return 0
        lax.fori_loop(0, _K // _L, gi_body, 0)
        gcps = [
            pltpu.async_copy(pc.at[gidx_v.at[pl.ds(c * _K, _K)]],
                             aout_v.at[pl.ds(c * _K, _K)], sem)
            for c in range(_D)
        ]
        for cp in gcps:
            cp.wait()
        pltpu.sync_copy(aout_v, a_out.at[pl.ds(b * _D * _K, _D * _K)])


def _sc_topk(dist, pcloud):
    mesh = plsc.VectorSubcoreMesh(core_axis_name="c", subcore_axis_name="s")
    f = pl.kernel(
        _sc_body,
        out_type=(
            jax.ShapeDtypeStruct((_B * _D * _K,), jnp.float32),
            jax.ShapeDtypeStruct((_B * _K,), jnp.int32),
        ),
        mesh=mesh,
        compiler_params=pltpu.CompilerParams(needs_layout_passes=False),
        scratch_types=[
            pltpu.VMEM((_N,), jnp.float32),
            pltpu.VMEM((_L * _BINS,), jnp.int32),
            pltpu.VMEM((_CAND + _L,), jnp.float32),
            pltpu.VMEM((_CAND + _L,), jnp.int32),
            pltpu.VMEM((_CAND + _L,), jnp.float32),
            pltpu.VMEM((_CAND + _L,), jnp.int32),
            pltpu.VMEM((_D * _K,), jnp.float32),
            pltpu.VMEM((_D * _K,), jnp.int32),
            pltpu.SemaphoreType.DMA,
        ],
    )
    zeros_h = jnp.zeros((_L * _BINS,), jnp.int32)
    padf_h = jnp.full((_CAND + _L,), jnp.inf, jnp.float32)
    padi_h = jnp.full((_CAND + _L,), _PADI, jnp.int32)
    a_f, b_f = f(dist.reshape(_B * _N), pcloud.reshape(_B * _D * _N),
                 zeros_h, padf_h, padi_h)
    return a_f.reshape(_B, _D, _K), b_f.reshape(_B, _K)


def kernel(pcloud, P1, K):
    dist = _norms(pcloud, P1).reshape(_B, _N)
    a, idx = _sc_topk(dist, pcloud)
    off = (jnp.asarray(K) - _K).astype(jnp.int32)
    return (a, idx + off)


# speedup vs baseline: 1.7068x; 1.4082x over previous
import jax
import jax.numpy as jnp
from jax import lax
from jax.experimental import pallas as pl
from jax.experimental.pallas import tpu as pltpu
from jax.experimental.pallas import tpu_sc as plsc

_B, _D, _N = 16, 3, 65536
_K = 1024
_L = 16
_BINS = 2048
_SHIFT = 20
_N2 = _N // 2
_NV2 = _N2 // _L
_CAND = 2048
_SORT = 1024
_PADI = 0x7FFFFFFF


def _norm_body(p1_ref, pc_ref, out_ref):
    b = pl.program_id(0)
    d = pc_ref[0]
    qx = p1_ref[b, 0]
    qy = p1_ref[b, 1]
    qz = p1_ref[b, 2]
    dx = d[0:1, :] - qx
    dy = d[1:2, :] - qy
    dz = d[2:3, :] - qz
    s = (dx * dx + dz * dz) + dy * dy
    out_ref[0] = jnp.sqrt(s)


def _norms(pcloud, P1):
    return pl.pallas_call(
        _norm_body,
        grid=(_B,),
        in_specs=[
            pl.BlockSpec((_B, _D), lambda b: (0, 0), memory_space=pltpu.SMEM),
            pl.BlockSpec((1, _D, _N), lambda b: (b, 0, 0)),
        ],
        out_specs=pl.BlockSpec((1, 1, _N), lambda b: (b, 0, 0)),
        out_shape=jax.ShapeDtypeStruct((_B, 1, _N), jnp.float32),
    )(P1, pcloud)


def _scalar(x):
    return jnp.max(x) if getattr(x, "ndim", 0) else x


def _sort_pairs(skey_v, sidx_v, n):
    iota = lax.iota(jnp.int32, _L)
    nvec = n // _L
    dn = lax.GatherDimensionNumbers(
        offset_dims=(), collapsed_slice_dims=(0,), start_index_map=(0,))
    k = 2
    while k <= n:
        j = k // 2
        while j >= 1:
            if j >= _L:
                dd = j // _L
                s = dd.bit_length() - 1

                def inter_body(u, _, k=k, dd=dd, s=s):
                    v_lo = ((u >> s) << (s + 1)) | (u & (dd - 1))
                    v_hi = v_lo + dd
                    ak = skey_v[pl.ds(v_lo * _L, _L)]
                    ai = sidx_v[pl.ds(v_lo * _L, _L)]
                    bk = skey_v[pl.ds(v_hi * _L, _L)]
                    bi = sidx_v[pl.ds(v_hi * _L, _L)]
                    asc = ((v_lo * _L) & k) == 0
                    altb = jnp.logical_or(
                        ak < bk, jnp.logical_and(ak == bk, ai < bi))
                    sel = altb == jnp.broadcast_to(asc, (_L,))
                    skey_v[pl.ds(v_lo * _L, _L)] = jnp.where(sel, ak, bk)
                    sidx_v[pl.ds(v_lo * _L, _L)] = jnp.where(sel, ai, bi)
                    skey_v[pl.ds(v_hi * _L, _L)] = jnp.where(sel, bk, ak)
                    sidx_v[pl.ds(v_hi * _L, _L)] = jnp.where(sel, bi, ai)
                    return 0
                lax.fori_loop(0, nvec // 2, inter_body, 0)
            else:
                perm = jnp.bitwise_xor(iota, j)
                is_hi = (iota & j) != 0

                def intra_body(v, _, k=k, j=j, perm=perm, is_hi=is_hi):
                    ak = skey_v[pl.ds(v * _L, _L)]
                    ai = sidx_v[pl.ds(v * _L, _L)]
                    bk = lax.gather(
                        ak, perm[:, None], dimension_numbers=dn,
                        slice_sizes=(1,),
                        mode=lax.GatherScatterMode.PROMISE_IN_BOUNDS)
                    bi = lax.gather(
                        ai, perm[:, None], dimension_numbers=dn,
                        slice_sizes=(1,),
                        mode=lax.GatherScatterMode.PROMISE_IN_BOUNDS)
                    if k <= 8:
                        asc = (iota & k) == 0
                    else:
                        asc = jnp.broadcast_to(((v * _L) & k) == 0, (_L,))
                    hold_min = asc != is_hi
                    altb = jnp.logical_or(
                        ak < bk, jnp.logical_and(ak == bk, ai < bi))
                    sel = altb == hold_min
                    skey_v[pl.ds(v * _L, _L)] = jnp.where(sel, ak, bk)
                    sidx_v[pl.ds(v * _L, _L)] = jnp.where(sel, ai, bi)
                    return 0
                lax.fori_loop(0, nvec, intra_body, 0)
            j //= 2
        k *= 2


def _sc_body(dist, pc, zeros_h, padf_h, padi_h, a_out, b_out,
             keys_v, hist_v, hist2_v, skey_v, sidx_v, bkey_v, bidx_v,
             cnt_v, aout_v, gidx_v,
             sh_hist, sh_dkey, sh_didx, sh_bkey, sh_bidx, sh_cnt, sh_srt,
             sem):
    cid = lax.axis_index("c")
    sub = lax.axis_index("s")
    wid = sub * 2 + cid
    batch = wid & 15
    half = lax.shift_right_logical(wid, 4)
    peer = jnp.bitwise_xor(sub, 8)
    iota = lax.iota(jnp.int32, _L)
    ones = jnp.ones((_L,), jnp.int32)

    cps = [
        pltpu.async_copy(dist.at[pl.ds(batch * _N + half * _N2, _N2)],
                         keys_v, sem),
        pltpu.async_copy(zeros_h, hist_v, sem),
        pltpu.async_copy(padf_h, skey_v, sem),
        pltpu.async_copy(padi_h, sidx_v, sem),
        pltpu.async_copy(padf_h, bkey_v, sem),
        pltpu.async_copy(padi_h, bidx_v, sem),
    ]
    for cp in cps:
        cp.wait()

    def hist_body(i, _):
        for u in range(8):
            kv = keys_v[pl.ds((i * 8 + u) * _L, _L)]
            bits = lax.bitcast_convert_type(kv, jnp.int32)
            binv = lax.shift_right_logical(bits, _SHIFT)
            addr = iota * _BINS + binv
            plsc.addupdate_scatter(hist_v, [addr], ones)
        return 0
    lax.fori_loop(0, _NV2 // 8, hist_body, 0)

    def red_body(j, _):
        acc = hist_v[pl.ds(j * _L, _L)]
        for r in range(1, _L):
            acc = acc + hist_v[pl.ds(r * _BINS + j * _L, _L)]
        hist_v[pl.ds(j * _L, _L)] = acc
        return 0
    lax.fori_loop(0, _BINS // _L, red_body, 0)

    pltpu.sync_copy(hist_v.at[pl.ds(0, _BINS)],
                    sh_hist.at[pl.ds(sub * _BINS, _BINS)])
    plsc.subcore_barrier()
    pltpu.sync_copy(sh_hist.at[pl.ds(peer * _BINS, _BINS)], hist2_v)

    def add_body(j, _):
        hist_v[pl.ds(j * _L, _L)] = (hist_v[pl.ds(j * _L, _L)]
                                     + hist2_v[pl.ds(j * _L, _L)])
        return 0
    lax.fori_loop(0, _BINS // _L, add_body, 0)

    def scan_body(j, carry):
        total, bstar, found = carry
        h16 = hist_v[pl.ds(j * _L, _L)]
        c16 = plsc.cumsum(h16)
        chunk = jnp.max(c16)
        cross = (total + c16) >= _K
        crossed = jnp.logical_and(total + chunk >= _K, found == 0)
        pos = _scalar(plsc.all_reduce_ffs(cross))
        bstar = jnp.where(crossed, j * _L + pos, bstar)
        found = jnp.where(crossed, 1, found)
        return (total + chunk, bstar, found)
    _, bstar, _ = lax.fori_loop(
        0, _BINS // _L, scan_body,
        (jnp.int32(0), jnp.int32(0), jnp.int32(0)))

    def comp_body(i, carry):
        nd, nb = carry
        kv = keys_v[pl.ds(i * _L, _L)]
        bits = lax.bitcast_convert_type(kv, jnp.int32)
        binv = lax.shift_right_logical(bits, _SHIFT)
        idx16 = half * _N2 + i * _L + iota
        mdef = binv < bstar
        mbor = binv == bstar
        plsc.store_compressed(skey_v.at[pl.ds(nd, _L)], kv, mask=mdef)
        plsc.store_compressed(sidx_v.at[pl.ds(nd, _L)], idx16, mask=mdef)
        plsc.store_compressed(bkey_v.at[pl.ds(nb, _L)], kv, mask=mbor)
        plsc.store_compressed(bidx_v.at[pl.ds(nb, _L)], idx16, mask=mbor)
        nd = nd + _scalar(plsc.all_reduce_population_count(mdef))
        nb = nb + _scalar(plsc.all_reduce_population_count(mbor))
        nd = jnp.minimum(nd, jnp.int32(_CAND - _L))
        nb = jnp.minimum(nb, jnp.int32(_CAND - _L))
        return (nd, nb)
    nd, nb = lax.fori_loop(0, _NV2, comp_body,
                           (jnp.int32(0), jnp.int32(0)))

    pltpu.sync_copy(skey_v.at[pl.ds(0, _CAND)],
                    sh_dkey.at[pl.ds(sub * _CAND, _CAND)])
    pltpu.sync_copy(sidx_v.at[pl.ds(0, _CAND)],
                    sh_didx.at[pl.ds(sub * _CAND, _CAND)])
    pltpu.sync_copy(bkey_v.at[pl.ds(0, _CAND)],
                    sh_bkey.at[pl.ds(sub * _CAND, _CAND)])
    pltpu.sync_copy(bidx_v.at[pl.ds(0, _CAND)],
                    sh_bidx.at[pl.ds(sub * _CAND, _CAND)])
    cnt_v[pl.ds(0, _L)] = jnp.broadcast_to(nd, (_L,))
    cnt_v[pl.ds(_L, _L)] = jnp.broadcast_to(nb, (_L,))
    pltpu.sync_copy(cnt_v, sh_cnt.at[pl.ds(sub * 2 * _L, 2 * _L)])
    plsc.subcore_barrier()

    pltpu.sync_copy(sh_cnt.at[pl.ds(peer * 2 * _L, 2 * _L)], cnt_v)
    nd_p = jnp.max(cnt_v[pl.ds(0, _L)])
    nb_p = jnp.max(cnt_v[pl.ds(_L, _L)])

    @pl.when(half == 1)
    def _():
        def mv_body(i, _):
            skey_v[pl.ds(i * _L, _L)] = bkey_v[pl.ds(i * _L, _L)]
            sidx_v[pl.ds(i * _L, _L)] = bidx_v[pl.ds(i * _L, _L)]
            return 0
        lax.fori_loop(0, (_SORT + _L) // _L, mv_body, 0)

    @pl.when(half == 0)
    def _():
        pltpu.sync_copy(sh_dkey.at[pl.ds(peer * _CAND, _CAND)],
                        bkey_v.at[pl.ds(0, _CAND)])
        pltpu.sync_copy(sh_didx.at[pl.ds(peer * _CAND, _CAND)],
                        bidx_v.at[pl.ds(0, _CAND)])

    @pl.when(half == 1)
    def _():
        pltpu.sync_copy(sh_bkey.at[pl.ds(peer * _CAND, _CAND)],
                        bkey_v.at[pl.ds(0, _CAND)])
        pltpu.sync_copy(sh_bidx.at[pl.ds(peer * _CAND, _CAND)],
                        bidx_v.at[pl.ds(0, _CAND)])

    n_own = jnp.where(half == 0, nd, nb)
    n_peer = jnp.where(half == 0, nd_p, nb_p)
    napp = jnp.minimum(n_peer, jnp.maximum(jnp.int32(0),
                                           jnp.int32(_SORT + _L) - n_own))

    def app_body(i, _):
        skey_v[pl.ds(n_own + i * _L, _L)] = bkey_v[pl.ds(i * _L, _L)]
        sidx_v[pl.ds(n_own + i * _L, _L)] = bidx_v[pl.ds(i * _L, _L)]
        return 0
    lax.fori_loop(0, (napp + _L - 1) // _L, app_body, 0)

    _sort_pairs(skey_v, sidx_v, _SORT)

    @pl.when(half == 1)
    def _():
        pltpu.sync_copy(sidx_v.at[pl.ds(0, _SORT)],
                        sh_srt.at[pl.ds(sub * _SORT, _SORT)])
    plsc.subcore_barrier()

    @pl.when(half == 0)
    def _():
        pltpu.sync_copy(sh_srt.at[pl.ds(peer * _SORT, _SORT)],
                        bidx_v.at[pl.ds(0, _SORT)])
        ndg = nd + nd_p
        rem = jnp.int32(_K) - ndg

        def app2_body(i, _):
            sidx_v[pl.ds(ndg + i * _L, _L)] = bidx_v[pl.ds(i * _L, _L)]
            return 0
        lax.fori_loop(0, (rem + _L - 1) // _L, app2_body, 0)

        pltpu.sync_copy(sidx_v.at[pl.ds(0, _K)],
                        b_out.at[pl.ds(batch * _K, _K)])

        def gi_body(v, _):
            si = sidx_v[pl.ds(v * _L, _L)]
            for c in range(_D):
                gidx_v[pl.ds(c * _K + v * _L, _L)] = (
                    si + (batch * _D + c) * _N)
            return 0
        lax.fori_loop(0, _K // _L, gi_body, 0)
        gcps = [
            pltpu.async_copy(pc.at[gidx_v.at[pl.ds(c * _K, _K)]],
                             aout_v.at[pl.ds(c * _K, _K)], sem)
            for c in range(_D)
        ]
        for cp in gcps:
            cp.wait()
        pltpu.sync_copy(aout_v,
                        a_out.at[pl.ds(batch * _D * _K, _D * _K)])


def _sc_topk(dist, pcloud):
    mesh = plsc.VectorSubcoreMesh(core_axis_name="c", subcore_axis_name="s")
    f = pl.kernel(
        _sc_body,
        out_type=(
            jax.ShapeDtypeStruct((_B * _D * _K,), jnp.float32),
            jax.ShapeDtypeStruct((_B * _K,), jnp.int32),
        ),
        mesh=mesh,
        compiler_params=pltpu.CompilerParams(needs_layout_passes=False),
        scratch_types=[
            pltpu.VMEM((_N2,), jnp.float32),
            pltpu.VMEM((_L * _BINS,), jnp.int32),
            pltpu.VMEM((_BINS,), jnp.int32),
            pltpu.VMEM((_CAND + _L,), jnp.float32),
            pltpu.VMEM((_CAND + _L,), jnp.int32),
            pltpu.VMEM((_CAND + _L,), jnp.float32),
            pltpu.VMEM((_CAND + _L,), jnp.int32),
            pltpu.VMEM((2 * _L,), jnp.int32),
            pltpu.VMEM((_D * _K,), jnp.float32),
            pltpu.VMEM((_D * _K,), jnp.int32),
            pltpu.VMEM_SHARED((16 * _BINS,), jnp.int32),
            pltpu.VMEM_SHARED((16 * _CAND,), jnp.float32),
            pltpu.VMEM_SHARED((16 * _CAND,), jnp.int32),
            pltpu.VMEM_SHARED((16 * _CAND,), jnp.float32),
            pltpu.VMEM_SHARED((16 * _CAND,), jnp.int32),
            pltpu.VMEM_SHARED((16 * 2 * _L,), jnp.int32),
            pltpu.VMEM_SHARED((16 * _SORT,), jnp.int32),
            pltpu.SemaphoreType.DMA,
        ],
    )
    zeros_h = jnp.zeros((_L * _BINS,), jnp.int32)
    padf_h = jnp.full((_CAND + _L,), jnp.inf, jnp.float32)
    padi_h = jnp.full((_CAND + _L,), _PADI, jnp.int32)
    a_f, b_f = f(dist.reshape(_B * _N), pcloud.reshape(_B * _D * _N),
                 zeros_h, padf_h, padi_h)
    return a_f.reshape(_B, _D, _K), b_f.reshape(_B, _K)


def kernel(pcloud, P1, K):
    dist = _norms(pcloud, P1).reshape(_B, _N)
    a, idx = _sc_topk(dist, pcloud)
    off = (jnp.asarray(K) - _K).astype(jnp.int32)
    return (a, idx + off)
